# baseline (speedup 1.0000x reference)
import functools
import math

import jax
import jax.numpy as jnp
from jax import lax
from jax.experimental import pallas as pl
from jax.experimental.pallas import tpu as pltpu
from jax.experimental.pallas import tpu_sc as plsc

VOCAB = 100000
SEQ = 2048
D_MODEL = 768
PAD_ID = 0
BATCH = 4

B_TOTAL = BATCH * SEQ
NC, NS = 2, 16
NW = NC * NS
CHUNK = 64

_SCALE = 1.0 / math.sqrt(D_MODEL)


def _sc_gather(table, idx_flat, n_rows):
    mesh = plsc.VectorSubcoreMesh(core_axis_name="c", subcore_axis_name="s")
    b_per_w = n_rows // NW
    n_chunks = b_per_w // CHUNK

    @functools.partial(
        pl.kernel,
        mesh=mesh,
        out_type=jax.ShapeDtypeStruct((n_rows, D_MODEL), jnp.float32),
        scratch_types=[
            pltpu.VMEM((b_per_w,), jnp.int32),
            pltpu.VMEM((CHUNK, D_MODEL), jnp.float32),
            pltpu.VMEM((CHUNK, D_MODEL), jnp.float32),
            pltpu.SemaphoreType.DMA,
            pltpu.SemaphoreType.DMA,
        ],
    )
    def k(table_hbm, idx_hbm, out_hbm, idx_v, rows_a, rows_b, sem_a, sem_b):
        wid = lax.axis_index("s") * NC + lax.axis_index("c")
        base = wid * b_per_w
        pltpu.sync_copy(idx_hbm.at[pl.ds(base, b_per_w)], idx_v)
        bufs = (rows_a, rows_b)
        sems = (sem_a, sem_b)
        pending = [None, None]
        for c in range(n_chunks):
            s = c & 1
            pending[s] = pltpu.async_copy(
                table_hbm.at[idx_v.at[pl.ds(c * CHUNK, CHUNK)]],
                bufs[s],
                sems[s],
            )
            if c >= 1:
                p = (c - 1) & 1
                pending[p].wait()
                pltpu.sync_copy(
                    bufs[p], out_hbm.at[pl.ds(base + (c - 1) * CHUNK, CHUNK)]
                )
        last = (n_chunks - 1) & 1
        pending[last].wait()
        pltpu.sync_copy(
            bufs[last],
            out_hbm.at[pl.ds(base + (n_chunks - 1) * CHUNK, CHUNK)],
        )

    return k(table, idx_flat)


def _tc_fuse_body(x_ref, g_ref, p_ref, o_ref):
    scale_row = jnp.where(x_ref[0, 0] != PAD_ID, _SCALE, 0.0)
    o_ref[0] = g_ref[0] * scale_row.reshape(SEQ, 1) + p_ref[...]


def _tc_fuse(gathered, x_part, pos_table, n_batch):
    g3 = gathered.reshape(n_batch, SEQ, D_MODEL)
    x3 = x_part.reshape(n_batch, 1, SEQ)
    return pl.pallas_call(
        _tc_fuse_body,
        grid=(n_batch,),
        in_specs=[
            pl.BlockSpec((1, 1, SEQ), lambda b: (b, 0, 0)),
            pl.BlockSpec((1, SEQ, D_MODEL), lambda b: (b, 0, 0)),
            pl.BlockSpec((SEQ, D_MODEL), lambda b: (0, 0)),
        ],
        out_specs=pl.BlockSpec((1, SEQ, D_MODEL), lambda b: (b, 0, 0)),
        out_shape=jax.ShapeDtypeStruct((n_batch, SEQ, D_MODEL), jnp.float32),
    )(x3, g3, pos_table)


_N_SPLITS = 2


def kernel(x, tok_table, pos_table):
    x_flat = x.astype(jnp.int32).reshape(B_TOTAL)
    rows_per = B_TOTAL // _N_SPLITS
    batch_per = BATCH // _N_SPLITS
    parts = []
    for h in range(_N_SPLITS):
        xh = lax.dynamic_slice(x_flat, (h * rows_per,), (rows_per,))
        gh = _sc_gather(tok_table, xh, rows_per)
        parts.append(_tc_fuse(gh, xh, pos_table, batch_per))
    return jnp.concatenate(parts, axis=0)


# speedup vs baseline: 1.3190x; 1.3190x over previous
import functools
import math

import jax
import jax.numpy as jnp
from jax import lax
from jax.experimental import pallas as pl
from jax.experimental.pallas import tpu as pltpu
from jax.experimental.pallas import tpu_sc as plsc

VOCAB = 100000
SEQ = 2048
D_MODEL = 768
PAD_ID = 0
BATCH = 4

B_TOTAL = BATCH * SEQ
NC, NS = 2, 16
NW = NC * NS
CHUNK = 64

_SCALE = 1.0 / math.sqrt(D_MODEL)


def _sc_gather(table, idx_flat, n_rows):
    mesh = plsc.VectorSubcoreMesh(core_axis_name="c", subcore_axis_name="s")
    b_per_w = n_rows // NW
    n_chunks = b_per_w // CHUNK

    @functools.partial(
        pl.kernel,
        mesh=mesh,
        out_type=jax.ShapeDtypeStruct((n_rows, D_MODEL), jnp.float32),
        scratch_types=[
            pltpu.VMEM((b_per_w,), jnp.int32),
            pltpu.VMEM((CHUNK, D_MODEL), jnp.float32),
            pltpu.VMEM((CHUNK, D_MODEL), jnp.float32),
            pltpu.SemaphoreType.DMA,
            pltpu.SemaphoreType.DMA,
        ],
    )
    def k(table_hbm, idx_hbm, out_hbm, idx_v, rows_a, rows_b, sem_a, sem_b):
        wid = lax.axis_index("s") * NC + lax.axis_index("c")
        base = wid * b_per_w
        pltpu.sync_copy(idx_hbm.at[pl.ds(base, b_per_w)], idx_v)
        bufs = (rows_a, rows_b)
        sems = (sem_a, sem_b)
        pending = [None, None]
        for c in range(n_chunks):
            s = c & 1
            pending[s] = pltpu.async_copy(
                table_hbm.at[idx_v.at[pl.ds(c * CHUNK, CHUNK)]],
                bufs[s],
                sems[s],
            )
            if c >= 1:
                p = (c - 1) & 1
                pending[p].wait()
                pltpu.sync_copy(
                    bufs[p], out_hbm.at[pl.ds(base + (c - 1) * CHUNK, CHUNK)]
                )
        last = (n_chunks - 1) & 1
        pending[last].wait()
        pltpu.sync_copy(
            bufs[last],
            out_hbm.at[pl.ds(base + (n_chunks - 1) * CHUNK, CHUNK)],
        )

    return k(table, idx_flat)


def _tc_fuse_body(x_ref, g_ref, p_ref, o_ref):
    scale_row = jnp.where(x_ref[0, 0] != PAD_ID, _SCALE, 0.0)
    o_ref[0] = g_ref[0] * scale_row.reshape(SEQ, 1) + p_ref[...]


def _tc_fuse(gathered, x_part, pos_table, n_batch):
    g3 = gathered.reshape(n_batch, SEQ, D_MODEL)
    x3 = x_part.reshape(n_batch, 1, SEQ)
    return pl.pallas_call(
        _tc_fuse_body,
        grid=(n_batch,),
        in_specs=[
            pl.BlockSpec((1, 1, SEQ), lambda b: (b, 0, 0)),
            pl.BlockSpec((1, SEQ, D_MODEL), lambda b: (b, 0, 0)),
            pl.BlockSpec((SEQ, D_MODEL), lambda b: (0, 0)),
        ],
        out_specs=pl.BlockSpec((1, SEQ, D_MODEL), lambda b: (b, 0, 0)),
        out_shape=jax.ShapeDtypeStruct((n_batch, SEQ, D_MODEL), jnp.float32),
    )(x3, g3, pos_table)


def kernel(x, tok_table, pos_table):
    x_flat = x.astype(jnp.int32).reshape(B_TOTAL)
    gathered = _sc_gather(tok_table, x_flat, B_TOTAL)
    return _tc_fuse(gathered, x_flat, pos_table, BATCH)


# speedup vs baseline: 1.3407x; 1.0165x over previous
import functools
import math

import jax
import jax.numpy as jnp
from jax import lax
from jax.experimental import pallas as pl
from jax.experimental.pallas import tpu as pltpu
from jax.experimental.pallas import tpu_sc as plsc

VOCAB = 100000
SEQ = 2048
D_MODEL = 768
PAD_ID = 0
BATCH = 4

B_TOTAL = BATCH * SEQ
NC, NS = 2, 16
NW = NC * NS
CHUNK = 32

_SCALE = 1.0 / math.sqrt(D_MODEL)


def _sc_gather(table, idx_flat, n_rows):
    mesh = plsc.VectorSubcoreMesh(core_axis_name="c", subcore_axis_name="s")
    b_per_w = n_rows // NW
    n_chunks = b_per_w // CHUNK
    nbuf = min(4, n_chunks)

    @functools.partial(
        pl.kernel,
        mesh=mesh,
        out_type=jax.ShapeDtypeStruct((n_rows, D_MODEL), jnp.float32),
        scratch_types=(
            [pltpu.VMEM((b_per_w,), jnp.int32)]
            + [pltpu.VMEM((CHUNK, D_MODEL), jnp.float32)] * nbuf
            + [pltpu.SemaphoreType.DMA] * nbuf
        ),
    )
    def k(table_hbm, idx_hbm, out_hbm, idx_v, *bufs_sems):
        bufs = bufs_sems[:nbuf]
        sems = bufs_sems[nbuf:]
        wid = lax.axis_index("s") * NC + lax.axis_index("c")
        base = wid * b_per_w
        pltpu.sync_copy(idx_hbm.at[pl.ds(base, b_per_w)], idx_v)
        pending = [None] * nbuf
        for c in range(n_chunks):
            s = c % nbuf
            pending[s] = pltpu.async_copy(
                table_hbm.at[idx_v.at[pl.ds(c * CHUNK, CHUNK)]],
                bufs[s],
                sems[s],
            )
            if c >= nbuf - 1:
                w = c - (nbuf - 1)
                p = w % nbuf
                pending[p].wait()
                pltpu.sync_copy(
                    bufs[p], out_hbm.at[pl.ds(base + w * CHUNK, CHUNK)]
                )
        for w in range(n_chunks - (nbuf - 1), n_chunks):
            p = w % nbuf
            pending[p].wait()
            pltpu.sync_copy(
                bufs[p], out_hbm.at[pl.ds(base + w * CHUNK, CHUNK)]
            )

    return k(table, idx_flat)


def _tc_fuse_body(x_ref, g_ref, p_ref, o_ref):
    scale_row = jnp.where(x_ref[0, 0] != PAD_ID, _SCALE, 0.0)
    o_ref[0] = g_ref[0] * scale_row.reshape(SEQ, 1) + p_ref[...]


def _tc_fuse(gathered, x_part, pos_table, n_batch):
    g3 = gathered.reshape(n_batch, SEQ, D_MODEL)
    x3 = x_part.reshape(n_batch, 1, SEQ)
    return pl.pallas_call(
        _tc_fuse_body,
        grid=(n_batch,),
        in_specs=[
            pl.BlockSpec((1, 1, SEQ), lambda b: (b, 0, 0)),
            pl.BlockSpec((1, SEQ, D_MODEL), lambda b: (b, 0, 0)),
            pl.BlockSpec((SEQ, D_MODEL), lambda b: (0, 0)),
        ],
        out_specs=pl.BlockSpec((1, SEQ, D_MODEL), lambda b: (b, 0, 0)),
        out_shape=jax.ShapeDtypeStruct((n_batch, SEQ, D_MODEL), jnp.float32),
    )(x3, g3, pos_table)


def kernel(x, tok_table, pos_table):
    x_flat = x.astype(jnp.int32).reshape(B_TOTAL)
    gathered = _sc_gather(tok_table, x_flat, B_TOTAL)
    return _tc_fuse(gathered, x_flat, pos_table, BATCH)
